# baseline (speedup 1.0000x reference)
import functools

import jax
import jax.numpy as jnp
from jax import lax
from jax.experimental import pallas as pl
from jax.experimental.pallas import tpu as pltpu
from jax.experimental.pallas import tpu_sc as plsc

D = 100
DP = 128
CHUNK = 128
NBUF = 5
NC = 2
NS = 16
NW = NC * NS


def _emb_kernel_body(table_hbm, idx_hbm, out_hbm, idx_v, rows_v, gsem, wsem):
    n_chunks = idx_hbm.shape[1]
    wid = lax.axis_index("s") * NC + lax.axis_index("c")
    pltpu.sync_copy(idx_hbm.at[wid], idx_v)

    def fire_gather(j, b):
        pltpu.async_copy(table_hbm.at[idx_v.at[j]], rows_v.at[b], gsem)

    def wait_gather(j, b):
        pltpu.make_async_copy(
            table_hbm.at[idx_v.at[j]], rows_v.at[b], gsem).wait()

    def fire_write(j, b):
        pltpu.async_copy(rows_v.at[b], out_hbm.at[wid, j], wsem)

    def wait_write(j, b):
        pltpu.make_async_copy(rows_v.at[b], out_hbm.at[wid, j], wsem).wait()

    def step(j, p, do_wait_w, do_fire_g):
        wait_gather(j, p)
        fire_write(j, p)
        if do_wait_w:
            wait_write(j - 1, (p - 1) % NBUF)
        if do_fire_g:
            fire_gather(j + NBUF - 1, (p - 1) % NBUF)

    for b in range(NBUF - 1):
        fire_gather(b, b)
    step(0, 0, False, True)
    for j in range(1, NBUF):
        step(j, j % NBUF, True, True)

    def body(g, carry):
        j0 = g * NBUF
        for p in range(NBUF):
            step(j0 + p, p, True, True)
        return carry

    lax.fori_loop(1, n_chunks // NBUF - 1, body, 0)

    j0 = n_chunks - NBUF
    for p in range(NBUF):
        j = j0 + p
        step(j, p, True, j + NBUF - 1 < n_chunks)
    wait_write(n_chunks - 1, (n_chunks - 1) % NBUF)


def kernel(word_embeddings, input_ids, seq_length):
    B0, S = input_ids.shape
    B = B0 * S
    n_chunks = B // (NW * CHUNK)
    idx = input_ids.reshape(-1).astype(jnp.int32).reshape(NW, n_chunks, CHUNK)
    table = jnp.pad(word_embeddings.astype(jnp.bfloat16), ((0, 0), (0, DP - D)))

    mesh = plsc.VectorSubcoreMesh(core_axis_name="c", subcore_axis_name="s")
    emb = functools.partial(
        pl.kernel,
        mesh=mesh,
        compiler_params=pltpu.CompilerParams(use_tc_tiling_on_sc=False),
        out_type=jax.ShapeDtypeStruct((NW, n_chunks, CHUNK, DP), jnp.bfloat16),
        scratch_types=[
            pltpu.VMEM((n_chunks, CHUNK), jnp.int32),
            pltpu.VMEM((NBUF, CHUNK, DP), jnp.bfloat16),
            pltpu.SemaphoreType.DMA,
            pltpu.SemaphoreType.DMA,
        ],
    )(_emb_kernel_body)

    out = emb(table, idx)
    sel = jnp.eye(DP, D, dtype=jnp.bfloat16)
    return jax.lax.dot_general(
        out.reshape(B, DP), sel, (((1,), (0,)), ((), ())),
        preferred_element_type=jnp.float32,
    ).reshape(B0, S, D)


# speedup vs baseline: 2.4016x; 2.4016x over previous
import functools

import jax
import jax.numpy as jnp
from jax import lax
from jax.experimental import pallas as pl
from jax.experimental.pallas import tpu as pltpu
from jax.experimental.pallas import tpu_sc as plsc

D = 100
DP = 128
CHUNK = 128
NBUF = 5
NC = 2
NS = 16
NW = NC * NS


def _emb_kernel_body(table_hbm, idx_hbm, out_hbm, idx_v, rows_v, gsem, wsem):
    n_chunks = idx_hbm.shape[1]
    wid = lax.axis_index("s") * NC + lax.axis_index("c")
    pltpu.sync_copy(idx_hbm.at[wid], idx_v)

    def fire_gather(j, b):
        pltpu.async_copy(table_hbm.at[idx_v.at[j]], rows_v.at[b], gsem)

    def wait_gather(j, b):
        pltpu.make_async_copy(
            table_hbm.at[idx_v.at[j]], rows_v.at[b], gsem).wait()

    def fire_write(j, b):
        pltpu.async_copy(rows_v.at[b], out_hbm.at[wid, j], wsem)

    def wait_write(j, b):
        pltpu.make_async_copy(rows_v.at[b], out_hbm.at[wid, j], wsem).wait()

    def step(j, p, do_wait_w, do_fire_g):
        wait_gather(j, p)
        fire_write(j, p)
        if do_wait_w:
            wait_write(j - 1, (p - 1) % NBUF)
        if do_fire_g:
            fire_gather(j + NBUF - 1, (p - 1) % NBUF)

    for b in range(NBUF - 1):
        fire_gather(b, b)
    step(0, 0, False, True)
    for j in range(1, NBUF):
        step(j, j % NBUF, True, True)

    def body(g, carry):
        j0 = g * NBUF
        for p in range(NBUF):
            step(j0 + p, p, True, True)
        return carry

    lax.fori_loop(1, n_chunks // NBUF - 1, body, 0)

    j0 = n_chunks - NBUF
    for p in range(NBUF):
        j = j0 + p
        step(j, p, True, j + NBUF - 1 < n_chunks)
    wait_write(n_chunks - 1, (n_chunks - 1) % NBUF)


def kernel(word_embeddings, input_ids, seq_length):
    B0, S = input_ids.shape
    B = B0 * S
    n_chunks = B // (NW * CHUNK)
    idx = input_ids.reshape(-1).astype(jnp.int32).reshape(NW, n_chunks, CHUNK)
    table = jnp.pad(word_embeddings, ((0, 0), (0, DP - D)))

    mesh = plsc.VectorSubcoreMesh(core_axis_name="c", subcore_axis_name="s")
    emb = functools.partial(
        pl.kernel,
        mesh=mesh,
        compiler_params=pltpu.CompilerParams(use_tc_tiling_on_sc=False),
        out_type=jax.ShapeDtypeStruct((NW, n_chunks, CHUNK, DP), jnp.float32),
        scratch_types=[
            pltpu.VMEM((n_chunks, CHUNK), jnp.int32),
            pltpu.VMEM((NBUF, CHUNK, DP), jnp.float32),
            pltpu.SemaphoreType.DMA,
            pltpu.SemaphoreType.DMA,
        ],
    )(_emb_kernel_body)

    out = emb(table, idx)
    return out.reshape(B0, S, DP)[:, :, :D]
